# baseline (speedup 1.0000x reference)
import functools

import jax
import jax.numpy as jnp
from jax import lax
from jax.experimental import pallas as pl
from jax.experimental.pallas import tpu as pltpu
from jax.experimental.pallas import tpu_sc as plsc


NB = 2


def _body(z_ref, e_ref, idx_ref):
    for j in range(NB):
        _one(z_ref, e_ref, idx_ref, j)


def _one(z_ref, e_ref, idx_ref, j):
    A = z_ref[j, 0]
    E = e_ref[0]
    K = E.shape[0]
    HW = A.shape[1]
    s = jax.lax.dot_general(E, A, (((1,), (0,)), ((), ())),
                            preferred_element_type=jnp.float32)
    en = jnp.sum(E * E, axis=1, keepdims=True)
    zn = jnp.sum(A * A, axis=0, keepdims=True)
    d2 = (zn + en) - 2.0 * s
    m1 = jnp.min(d2, axis=0, keepdims=True)
    s0 = jnp.sqrt(jnp.maximum(m1, 0.0))
    mbits = jax.lax.bitcast_convert_type(m1, jnp.int32)
    T = m1
    for i in (1, 2, 3):
        ci = jax.lax.bitcast_convert_type(mbits + i, jnp.float32)
        si = jnp.sqrt(jnp.maximum(ci, 0.0))
        T = jnp.where(si == s0, ci, T)
    T = jnp.where(s0 == 0.0, 0.0, T)
    idx = jnp.argmin(jnp.maximum(d2, T), axis=0).astype(jnp.int32)
    idx_ref[0, j] = idx.reshape(idx_ref.shape[2], idx_ref.shape[3])


def _tc_indices(z, e):
    N, ZD, H, W = z.shape
    L, K, C = e.shape
    HW = H * W
    zr = z.reshape(N, L, C, HW)
    return pl.pallas_call(
        _body,
        grid=(L, N // NB),
        in_specs=[
            pl.BlockSpec((NB, 1, C, HW), lambda l, n: (n, l, 0, 0)),
            pl.BlockSpec((1, K, C), lambda l, n: (l, 0, 0)),
        ],
        out_specs=pl.BlockSpec((1, NB, 8, HW // 8), lambda l, n: (l, n, 0, 0)),
        out_shape=jax.ShapeDtypeStruct((L, N, 8, HW // 8), jnp.int32),
        compiler_params=pltpu.CompilerParams(
            dimension_semantics=("parallel", "parallel")),
    )(zr, e)


def _sc_gather(table, flat_idx, D):
    B = flat_idx.shape[0]
    info = plsc.get_sparse_core_info()
    NC, NS = info.num_cores, info.num_subcores
    NW = NC * NS
    b_per_w = B // NW
    CH = 512
    mesh = plsc.VectorSubcoreMesh(core_axis_name="c", subcore_axis_name="s")

    @functools.partial(
        pl.kernel, mesh=mesh,
        out_type=jax.ShapeDtypeStruct((B, D), jnp.float32),
        scratch_types=[
            pltpu.VMEM((CH,), jnp.int32),
            pltpu.VMEM((CH, D), jnp.float32),
            pltpu.SemaphoreType.DMA,
        ],
    )
    def k(table_hbm, idx_hbm, out_hbm, idx_v, rows_v, sem):
        wid = lax.axis_index("s") * NC + lax.axis_index("c")
        base = wid * b_per_w
        for ci in range(b_per_w // CH):
            off = base + ci * CH
            pltpu.sync_copy(idx_hbm.at[pl.ds(off, CH)], idx_v)
            pltpu.async_copy(table_hbm.at[idx_v], rows_v, sem).wait()
            pltpu.sync_copy(rows_v, out_hbm.at[pl.ds(off, CH)])

    return k(table, flat_idx)


def kernel(z, e):
    N, ZD, H, W = z.shape
    L, K, C = e.shape
    HW = H * W
    idx = _tc_indices(z, e)
    flat = (idx + (jnp.arange(L, dtype=jnp.int32) * K).reshape(L, 1, 1, 1))
    e_pad = jnp.pad(e.reshape(L * K, C), ((0, 0), (0, 128 - C)))
    rows = _sc_gather(e_pad, flat.reshape(L * N * HW), 128)
    zq = (rows.reshape(L, N, HW, 128)[..., :C]
          .transpose(1, 0, 3, 2).reshape(N, ZD, H, W))
    z_out = z + (zq - z)
    return z_out, idx.reshape(L, N, H, W)


# speedup vs baseline: 1.3244x; 1.3244x over previous
import jax
import jax.numpy as jnp
from jax.experimental import pallas as pl
from jax.experimental.pallas import tpu as pltpu


NB = 4


def _body(z_ref, e_ref, zo_ref, idx_ref):
    for j in range(NB):
        _one(z_ref, e_ref, zo_ref, idx_ref, j)


def _one(z_ref, e_ref, zo_ref, idx_ref, j):
    A = z_ref[j, 0]
    E = e_ref[0]
    K = E.shape[0]
    HW = A.shape[1]
    s = jax.lax.dot_general(E, A, (((1,), (0,)), ((), ())),
                            preferred_element_type=jnp.float32)
    en = jnp.sum(E * E, axis=1, keepdims=True)
    zn = jnp.sum(A * A, axis=0, keepdims=True)
    d2 = (zn + en) - 2.0 * s
    m1 = jnp.min(d2, axis=0, keepdims=True)
    s0 = jnp.sqrt(jnp.maximum(m1, 0.0))
    mbits = jax.lax.bitcast_convert_type(m1, jnp.int32)
    T = m1
    for i in (1, 2, 3):
        ci = jax.lax.bitcast_convert_type(mbits + i, jnp.float32)
        si = jnp.sqrt(jnp.maximum(ci, 0.0))
        T = jnp.where(si == s0, ci, T)
    T = jnp.where(s0 == 0.0, 0.0, T)
    idx = jnp.argmin(jnp.maximum(d2, T), axis=0).astype(jnp.int32)
    kio = jax.lax.broadcasted_iota(jnp.int32, (K, HW), 0)
    oh = (kio == idx[None, :]).astype(jnp.float32)
    zq = jax.lax.dot_general(E, oh, (((0,), (0,)), ((), ())),
                             preferred_element_type=jnp.float32)
    zo_ref[j, 0] = A + (zq - A)
    idx_ref[0, j] = idx.reshape(idx_ref.shape[2], idx_ref.shape[3])


def kernel(z, e):
    N, ZD, H, W = z.shape
    L, K, C = e.shape
    HW = H * W
    zr = z.reshape(N, L, C, HW)
    zo, idx = pl.pallas_call(
        _body,
        grid=(L, N // NB),
        in_specs=[
            pl.BlockSpec((NB, 1, C, HW), lambda l, n: (n, l, 0, 0)),
            pl.BlockSpec((1, K, C), lambda l, n: (l, 0, 0)),
        ],
        out_specs=[
            pl.BlockSpec((NB, 1, C, HW), lambda l, n: (n, l, 0, 0)),
            pl.BlockSpec((1, NB, 8, HW // 8), lambda l, n: (l, n, 0, 0)),
        ],
        out_shape=[
            jax.ShapeDtypeStruct((N, L, C, HW), jnp.float32),
            jax.ShapeDtypeStruct((L, N, 8, HW // 8), jnp.int32),
        ],
        compiler_params=pltpu.CompilerParams(
            dimension_semantics=("parallel", "parallel")),
    )(zr, e)
    return zo.reshape(N, ZD, H, W), idx.reshape(L, N, H, W)
